# baseline (speedup 1.0000x reference)
import functools

import jax
import jax.numpy as jnp
from jax.experimental import pallas as pl
from jax.experimental.pallas import tpu as pltpu

BF = jnp.bfloat16
F32 = jnp.float32


def _fused_cnn_kernel(xv, wb1, wb2, s1m, s2m, w3b, w4b, b1, b2, b3, b4,
                      out_ref, sxm, sp1r0, sp1r1, sp1r2, sp1r3, sp2, *, TB):
    M4 = TB * 4

    xcv = jnp.concatenate([xv[:, c, :, :] for c in range(3)],
                          axis=2)
    xcv = jnp.pad(xcv, ((0, 0), (0, 0), (0, 32)))
    vt = jnp.transpose(xcv.reshape(TB, 4, 8, 128), (0, 2, 1, 3))
    sxm[:, 1:8, 0:4, :] = vt[:, 0:7, :, :]
    sxm[:, 0, 1:5, :] = vt[:, 7, 0:4, :]
    sxm[:, 0, 0, :] = jnp.zeros((TB, 128), F32)
    sxm[:, 1:8, 4, :] = jnp.zeros((TB, 7, 128), F32)

    def xp(r, q0):
        return sxm[:, r, q0:q0 + 4, :].reshape(M4, 128)

    def blk(r):
        pieces = [xp((r + d) % 8, 1 if r + d >= 8 else 0) for d in range(3)]
        return jnp.concatenate(pieces, axis=1)

    targets = ((sp1r1, 0), (sp1r2, 0), (sp1r3, 0), (sp1r0, 1))
    for k in range(4):
        xck = jnp.concatenate([blk(2 * k), blk(2 * k + 1)],
                              axis=0).astype(BF)
        a1k = jnp.dot(xck, wb1[...], preferred_element_type=F32)
        a1k = jnp.maximum(a1k + b1[...], 0.0)
        hpk = jnp.maximum(a1k[:M4], a1k[M4:])
        vrk = jnp.concatenate([hpk[:, 1:], hpk[:, :1]], axis=1)
        m1k = jnp.maximum(hpk, vrk).astype(BF)
        p1k = jnp.dot(m1k, s1m[...], preferred_element_type=F32).astype(BF)
        ref, q0 = targets[k]
        ref[:, q0:q0 + 4, :] = p1k.reshape(TB, 4, 512)
    sp1r0[:, 0, :] = jnp.zeros((TB, 512), BF)
    sp1r1[:, 4, :] = jnp.zeros((TB, 512), BF)

    def sp(ref, q0):
        return ref[:, q0:q0 + 4, :].reshape(M4, 512)

    L0 = jnp.concatenate([sp(sp1r0, 0), sp(sp1r1, 0), sp(sp1r2, 0)], axis=1)
    L1 = jnp.concatenate([sp(sp1r1, 0), sp(sp1r2, 0), sp(sp1r3, 0)], axis=1)
    L2 = jnp.concatenate([sp(sp1r2, 0), sp(sp1r3, 0), sp(sp1r0, 1)], axis=1)
    L3 = jnp.concatenate([sp(sp1r3, 0), sp(sp1r0, 1), sp(sp1r1, 1)], axis=1)
    for k, (lo, hi) in enumerate(((L0, L1), (L2, L3))):
        yck = jnp.concatenate([lo, hi], axis=0)
        a2k = jnp.dot(yck, wb2[...], preferred_element_type=F32)
        a2k = jnp.maximum(a2k + b2[...], 0.0)
        hp2k = jnp.maximum(a2k[:M4], a2k[M4:])
        vr2k = jnp.concatenate([hp2k[:, 64:], hp2k[:, :64]], axis=1)
        m2k = jnp.maximum(hp2k, vr2k).astype(BF)
        p2k = jnp.dot(m2k, s2m[...], preferred_element_type=F32).astype(BF)
        sp2[:, 4 * k:4 * k + 4, :] = p2k.reshape(TB, 4, 512)

    h1 = b3[...].astype(F32)
    for s, h3 in enumerate((0, 2, 4, 6, 1, 3, 5, 7)):
        h1 = h1 + jnp.dot(sp2[:, s, :], w3b[h3],
                          preferred_element_type=F32)
    h1 = jnp.maximum(h1, 0.0).astype(BF)
    o = jnp.dot(h1, w4b[...], preferred_element_type=F32) + b4[...]
    out_ref[...] = o[:, :10].astype(F32)


def kernel(w1p, b1p, w2p, b2p, w3p, b3p, w4p, b4p, x_nchw):
    B = x_nchw.shape[0]
    TB = 64 if x_nchw.shape[0] % 64 == 0 else 16

    w1r = w1p.reshape(3, 3, 3, 128)[:, :, :, :32]
    w2r = w2p.reshape(3, 3, 128, 128)[:, :, :32, :64]
    eyes32 = jnp.stack([jnp.eye(32, k=1 - dw, dtype=F32) for dw in range(3)])
    eyes16 = jnp.stack([jnp.eye(16, k=1 - dw, dtype=F32) for dw in range(3)])

    wb1 = (w1r[:, :, :, None, :, None] *
           eyes32[None, :, None, :, None, :]).sum(axis=1)
    wb1 = wb1.reshape(3, 96, 1024)
    wb1 = jnp.pad(wb1, ((0, 0), (0, 32), (0, 0))).reshape(384, 1024).astype(BF)
    wb2 = (w2r[:, :, :, None, None, :] *
           eyes16[None, :, None, :, :, None]).sum(axis=1)
    wb2 = wb2.reshape(1536, 1024).astype(BF)

    ii = jax.lax.broadcasted_iota(jnp.int32, (1024, 512), 0)
    jj = jax.lax.broadcasted_iota(jnp.int32, (1024, 512), 1)
    s1m = (ii == (jj // 16) * 32 + (jj % 16) * 2).astype(BF)
    s2m = (ii == (jj // 64) * 128 + jj % 64).astype(BF)

    w3b = w3p.reshape(8, 8, 128, 128)[:, :, :64, :]
    w3b = w3b.reshape(8, 512, 128).astype(BF)
    w4b = w4p.astype(BF)

    b1bc = jnp.repeat(b1p.reshape(-1)[:32], 32).reshape(1, 1024)
    b2bc = jnp.tile(b2p.reshape(-1)[:64], 16).reshape(1, 1024)

    xv = x_nchw

    kern = functools.partial(_fused_cnn_kernel, TB=TB)
    out = pl.pallas_call(
        kern,
        out_shape=jax.ShapeDtypeStruct((B, 10), F32),
        grid=(B // TB,),
        in_specs=[
            pl.BlockSpec((TB, 3, 32, 32), lambda i: (i, 0, 0, 0)),
            pl.BlockSpec((384, 1024), lambda i: (0, 0)),
            pl.BlockSpec((1536, 1024), lambda i: (0, 0)),
            pl.BlockSpec((1024, 512), lambda i: (0, 0)),
            pl.BlockSpec((1024, 512), lambda i: (0, 0)),
            pl.BlockSpec((8, 512, 128), lambda i: (0, 0, 0)),
            pl.BlockSpec((128, 128), lambda i: (0, 0)),
            pl.BlockSpec((1, 1024), lambda i: (0, 0)),
            pl.BlockSpec((1, 1024), lambda i: (0, 0)),
            pl.BlockSpec((1, 128), lambda i: (0, 0)),
            pl.BlockSpec((1, 128), lambda i: (0, 0)),
        ],
        out_specs=pl.BlockSpec((TB, 10), lambda i: (i, 0)),
        scratch_shapes=[
            pltpu.VMEM((TB, 8, 5, 128), F32),
            pltpu.VMEM((TB, 5, 512), BF),
            pltpu.VMEM((TB, 5, 512), BF),
            pltpu.VMEM((TB, 4, 512), BF),
            pltpu.VMEM((TB, 4, 512), BF),
            pltpu.VMEM((TB, 8, 512), BF),
        ],
        compiler_params=pltpu.CompilerParams(
            dimension_semantics=("parallel",)),
    )(xv, wb1, wb2, s1m, s2m, w3b, w4b, b1bc, b2bc, b3p, b4p)
    return out


# speedup vs baseline: 1.1862x; 1.1862x over previous
import functools

import jax
import jax.numpy as jnp
from jax.experimental import pallas as pl
from jax.experimental.pallas import tpu as pltpu

BF = jnp.bfloat16
F32 = jnp.float32


def _fused_cnn_kernel(xv, wb1, wb2, s1m, s2m, w3b, w4b, b1, b2, b3, b4,
                      out_ref, sxm, sp1r0, sp1r1, sp1r2, sp1r3, sp2, *, TB):
    M4 = TB * 4

    xcv = jnp.concatenate([xv[:, 32 * c:32 * c + 32, :] for c in range(3)],
                          axis=2)
    xcv = jnp.pad(xcv, ((0, 0), (0, 0), (0, 32)))
    vt = jnp.transpose(xcv.reshape(TB, 4, 8, 128), (0, 2, 1, 3))
    sxm[:, 1:8, 0:4, :] = vt[:, 0:7, :, :]
    sxm[:, 0, 1:5, :] = vt[:, 7, 0:4, :]
    sxm[:, 0, 0, :] = jnp.zeros((TB, 128), F32)
    sxm[:, 1:8, 4, :] = jnp.zeros((TB, 7, 128), F32)

    def xp(r, q0):
        return sxm[:, r, q0:q0 + 4, :].reshape(M4, 128)

    blocks = []
    for r in range(8):
        pieces = [xp((r + d) % 8, 1 if r + d >= 8 else 0) for d in range(3)]
        blocks.append(jnp.concatenate(pieces, axis=1))
    xcat = jnp.concatenate(blocks, axis=0).astype(BF)
    a1 = jnp.dot(xcat, wb1[...], preferred_element_type=F32)
    a1 = jnp.maximum(a1 + b1[...], 0.0)

    hp = jnp.concatenate(
        [jnp.maximum(a1[(2 * k) * M4:(2 * k + 1) * M4],
                     a1[(2 * k + 1) * M4:(2 * k + 2) * M4]) for k in range(4)],
        axis=0)
    vr = jnp.concatenate([hp[:, 1:], hp[:, :1]], axis=1)
    m1 = jnp.maximum(hp, vr).astype(BF)
    p1 = jnp.dot(m1, s1m[...], preferred_element_type=F32).astype(BF)

    sp1r1[:, 0:4, :] = p1[0 * M4:1 * M4].reshape(TB, 4, 512)
    sp1r2[:, 0:4, :] = p1[1 * M4:2 * M4].reshape(TB, 4, 512)
    sp1r3[:, 0:4, :] = p1[2 * M4:3 * M4].reshape(TB, 4, 512)
    sp1r0[:, 1:5, :] = p1[3 * M4:4 * M4].reshape(TB, 4, 512)
    sp1r0[:, 0, :] = jnp.zeros((TB, 512), BF)
    sp1r1[:, 4, :] = jnp.zeros((TB, 512), BF)

    def sp(ref, q0):
        return ref[:, q0:q0 + 4, :].reshape(M4, 512)

    L0 = jnp.concatenate([sp(sp1r0, 0), sp(sp1r1, 0), sp(sp1r2, 0)], axis=1)
    L1 = jnp.concatenate([sp(sp1r1, 0), sp(sp1r2, 0), sp(sp1r3, 0)], axis=1)
    L2 = jnp.concatenate([sp(sp1r2, 0), sp(sp1r3, 0), sp(sp1r0, 1)], axis=1)
    L3 = jnp.concatenate([sp(sp1r3, 0), sp(sp1r0, 1), sp(sp1r1, 1)], axis=1)
    ycat = jnp.concatenate([L0, L1, L2, L3], axis=0)
    a2 = jnp.dot(ycat, wb2[...], preferred_element_type=F32)
    a2 = jnp.maximum(a2 + b2[...], 0.0)

    hp2 = jnp.concatenate(
        [jnp.maximum(a2[(2 * k) * M4:(2 * k + 1) * M4],
                     a2[(2 * k + 1) * M4:(2 * k + 2) * M4]) for k in range(2)],
        axis=0)
    vr2 = jnp.concatenate([hp2[:, 64:], hp2[:, :64]], axis=1)
    m2 = jnp.maximum(hp2, vr2).astype(BF)
    p2 = jnp.dot(m2, s2m[...], preferred_element_type=F32).astype(BF)
    sp2[:, 0:4, :] = p2[:M4].reshape(TB, 4, 512)
    sp2[:, 4:8, :] = p2[M4:].reshape(TB, 4, 512)

    h1 = b3[...].astype(F32)
    for s, h3 in enumerate((0, 2, 4, 6, 1, 3, 5, 7)):
        h1 = h1 + jnp.dot(sp2[:, s, :], w3b[h3],
                          preferred_element_type=F32)
    h1 = jnp.maximum(h1, 0.0).astype(BF)
    o = jnp.dot(h1, w4b[...], preferred_element_type=F32) + b4[...]
    out_ref[...] = o.astype(F32)


def kernel(w1p, b1p, w2p, b2p, w3p, b3p, w4p, b4p, x_nchw):
    B = x_nchw.shape[0]
    TB = 64 if x_nchw.shape[0] % 64 == 0 else 16

    w1r = w1p.reshape(3, 3, 3, 128)[:, :, :, :32]
    w2r = w2p.reshape(3, 3, 128, 128)[:, :, :32, :64]
    eyes32 = jnp.stack([jnp.eye(32, k=1 - dw, dtype=F32) for dw in range(3)])
    eyes16 = jnp.stack([jnp.eye(16, k=1 - dw, dtype=F32) for dw in range(3)])

    wb1 = jnp.einsum("hdic,dwv->hiwcv", w1r, eyes32).reshape(3, 96, 1024)
    wb1 = jnp.pad(wb1, ((0, 0), (0, 32), (0, 0))).reshape(384, 1024).astype(BF)
    wb2 = jnp.einsum("hdic,dwv->hiwvc", w2r, eyes16).reshape(3, 512, 1024)
    wb2 = wb2.reshape(1536, 1024).astype(BF)

    ii = jax.lax.broadcasted_iota(jnp.int32, (1024, 512), 0)
    jj = jax.lax.broadcasted_iota(jnp.int32, (1024, 512), 1)
    s1m = (ii == (jj // 16) * 32 + (jj % 16) * 2).astype(BF)
    s2m = (ii == (jj // 64) * 128 + jj % 64).astype(BF)

    w3b = w3p.reshape(8, 8, 128, 128)[:, :, :64, :]
    w3b = w3b.reshape(8, 512, 128).astype(BF)
    w4b = w4p.astype(BF)

    b1bc = jnp.repeat(b1p.reshape(-1)[:32], 32).reshape(1, 1024)
    b2bc = jnp.tile(b2p.reshape(-1)[:64], 16).reshape(1, 1024)

    xv = x_nchw.reshape(B, 96, 32)

    kern = functools.partial(_fused_cnn_kernel, TB=TB)
    out = pl.pallas_call(
        kern,
        out_shape=jax.ShapeDtypeStruct((B, 128), F32),
        grid=(B // TB,),
        in_specs=[
            pl.BlockSpec((TB, 96, 32), lambda i: (i, 0, 0)),
            pl.BlockSpec((384, 1024), lambda i: (0, 0)),
            pl.BlockSpec((1536, 1024), lambda i: (0, 0)),
            pl.BlockSpec((1024, 512), lambda i: (0, 0)),
            pl.BlockSpec((1024, 512), lambda i: (0, 0)),
            pl.BlockSpec((8, 512, 128), lambda i: (0, 0, 0)),
            pl.BlockSpec((128, 128), lambda i: (0, 0)),
            pl.BlockSpec((1, 1024), lambda i: (0, 0)),
            pl.BlockSpec((1, 1024), lambda i: (0, 0)),
            pl.BlockSpec((1, 128), lambda i: (0, 0)),
            pl.BlockSpec((1, 128), lambda i: (0, 0)),
        ],
        out_specs=pl.BlockSpec((TB, 128), lambda i: (i, 0)),
        scratch_shapes=[
            pltpu.VMEM((TB, 8, 5, 128), F32),
            pltpu.VMEM((TB, 5, 512), BF),
            pltpu.VMEM((TB, 5, 512), BF),
            pltpu.VMEM((TB, 4, 512), BF),
            pltpu.VMEM((TB, 4, 512), BF),
            pltpu.VMEM((TB, 8, 512), BF),
        ],
        compiler_params=pltpu.CompilerParams(
            dimension_semantics=("parallel",)),
    )(xv, wb1, wb2, s1m, s2m, w3b, w4b, b1bc, b2bc, b3p, b4p)
    return out[:, :10]


# speedup vs baseline: 1.2068x; 1.0174x over previous
import functools

import jax
import jax.numpy as jnp
from jax.experimental import pallas as pl
from jax.experimental.pallas import tpu as pltpu

BF = jnp.bfloat16
F32 = jnp.float32


def _fused_cnn_kernel(xv, wb1, wb2, s1m, s2m, w3b, w4b, b1, b2, b3, b4,
                      out_ref, sxm, sp1r0, sp1r1, sp1r2, sp1r3, sp2, *, TB):
    M4 = TB * 4

    xcv = jnp.concatenate([xv[:, 32 * c:32 * c + 32, :] for c in range(3)],
                          axis=2)
    xcv = jnp.concatenate(
        [xcv, jnp.ones((TB, 32, 1), F32), jnp.zeros((TB, 32, 31), F32)],
        axis=2)
    vt = jnp.transpose(xcv.reshape(TB, 4, 8, 128), (0, 2, 1, 3))
    sxm[:, 1:8, 0:4, :] = vt[:, 0:7, :, :]
    sxm[:, 0, 1:5, :] = vt[:, 7, 0:4, :]
    sxm[:, 0, 0, :] = jnp.zeros((TB, 128), F32)
    sxm[:, 1:8, 4, :] = jnp.zeros((TB, 7, 128), F32)

    def xp(r, q0):
        return sxm[:, r, q0:q0 + 4, :].reshape(M4, 128)

    blocks = []
    for r in range(8):
        pieces = [xp((r + d) % 8, 1 if r + d >= 8 else 0) for d in range(3)]
        blocks.append(jnp.concatenate(pieces, axis=1))
    xcat = jnp.concatenate(blocks, axis=0).astype(BF)
    a1 = jnp.dot(xcat, wb1[...], preferred_element_type=F32)
    a1 = jnp.maximum(a1, 0.0)

    hp = jnp.concatenate(
        [jnp.maximum(a1[(2 * k) * M4:(2 * k + 1) * M4],
                     a1[(2 * k + 1) * M4:(2 * k + 2) * M4]) for k in range(4)],
        axis=0)
    vr = jnp.concatenate([hp[:, 1:], hp[:, :1]], axis=1)
    m1 = jnp.maximum(hp, vr).astype(BF)
    p1 = jnp.dot(m1, s1m[...], preferred_element_type=F32).astype(BF)

    sp1r1[:, 0:4, :] = p1[0 * M4:1 * M4].reshape(TB, 4, 512)
    sp1r2[:, 0:4, :] = p1[1 * M4:2 * M4].reshape(TB, 4, 512)
    sp1r3[:, 0:4, :] = p1[2 * M4:3 * M4].reshape(TB, 4, 512)
    sp1r0[:, 1:5, :] = p1[3 * M4:4 * M4].reshape(TB, 4, 512)
    sp1r0[:, 0, :] = jnp.zeros((TB, 512), BF)
    sp1r1[:, 4, :] = jnp.zeros((TB, 512), BF)

    def sp(ref, q0):
        return ref[:, q0:q0 + 4, :].reshape(M4, 512)

    L0 = jnp.concatenate([sp(sp1r0, 0), sp(sp1r1, 0), sp(sp1r2, 0)], axis=1)
    L1 = jnp.concatenate([sp(sp1r1, 0), sp(sp1r2, 0), sp(sp1r3, 0)], axis=1)
    L2 = jnp.concatenate([sp(sp1r2, 0), sp(sp1r3, 0), sp(sp1r0, 1)], axis=1)
    L3 = jnp.concatenate([sp(sp1r3, 0), sp(sp1r0, 1), sp(sp1r1, 1)], axis=1)
    ycat = jnp.concatenate([L0, L1, L2, L3], axis=0)
    a2 = jnp.dot(ycat, wb2[...], preferred_element_type=F32)
    a2 = jnp.maximum(a2 + b2[...], 0.0)

    hp2 = jnp.concatenate(
        [jnp.maximum(a2[(2 * k) * M4:(2 * k + 1) * M4],
                     a2[(2 * k + 1) * M4:(2 * k + 2) * M4]) for k in range(2)],
        axis=0)
    vr2 = jnp.concatenate([hp2[:, 64:], hp2[:, :64]], axis=1)
    m2 = jnp.maximum(hp2, vr2).astype(BF)
    p2 = jnp.dot(m2, s2m[...], preferred_element_type=F32).astype(BF)
    sp2[:, 0:4, :] = p2[:M4].reshape(TB, 4, 512)
    sp2[:, 4:8, :] = p2[M4:].reshape(TB, 4, 512)

    h1 = b3[...].astype(F32)
    for s, h3 in enumerate((0, 2, 4, 6, 1, 3, 5, 7)):
        h1 = h1 + jnp.dot(sp2[:, s, :], w3b[h3],
                          preferred_element_type=F32)
    h1 = jnp.maximum(h1, 0.0).astype(BF)
    o = jnp.dot(h1, w4b[...], preferred_element_type=F32) + b4[...]
    out_ref[...] = o.astype(F32)


def kernel(w1p, b1p, w2p, b2p, w3p, b3p, w4p, b4p, x_nchw):
    B = x_nchw.shape[0]
    TB = 64 if x_nchw.shape[0] % 64 == 0 else 16

    w1r = w1p.reshape(3, 3, 3, 128)[:, :, :, :32]
    w2r = w2p.reshape(3, 3, 128, 128)[:, :, :32, :64]
    eyes32 = jnp.stack([jnp.eye(32, k=1 - dw, dtype=F32) for dw in range(3)])
    eyes16 = jnp.stack([jnp.eye(16, k=1 - dw, dtype=F32) for dw in range(3)])

    wb1 = jnp.einsum("hdic,dwv->hiwcv", w1r, eyes32).reshape(3, 96, 1024)
    wb1 = jnp.pad(wb1, ((0, 0), (0, 32), (0, 0)))
    b1bc = jnp.repeat(b1p.reshape(-1)[:32], 32).reshape(1, 1, 1024)
    wb1 = jnp.concatenate(
        [wb1[0:1], jnp.concatenate([wb1[1:2, :96], b1bc, wb1[1:2, 97:]],
                                   axis=1), wb1[2:3]], axis=0)
    wb1 = wb1.reshape(384, 1024).astype(BF)
    wb2 = jnp.einsum("hdic,dwv->hiwvc", w2r, eyes16).reshape(3, 512, 1024)
    wb2 = wb2.reshape(1536, 1024).astype(BF)

    ii = jax.lax.broadcasted_iota(jnp.int32, (1024, 512), 0)
    jj = jax.lax.broadcasted_iota(jnp.int32, (1024, 512), 1)
    s1m = (ii == (jj // 16) * 32 + (jj % 16) * 2).astype(BF)
    s2m = (ii == (jj // 64) * 128 + jj % 64).astype(BF)

    w3b = w3p.reshape(8, 8, 128, 128)[:, :, :64, :]
    w3b = w3b.reshape(8, 512, 128).astype(BF)
    w4b = w4p.astype(BF)

    b1bc = jnp.repeat(b1p.reshape(-1)[:32], 32).reshape(1, 1024)
    b2bc = jnp.tile(b2p.reshape(-1)[:64], 16).reshape(1, 1024)

    xv = x_nchw.reshape(B, 96, 32)

    kern = functools.partial(_fused_cnn_kernel, TB=TB)
    out = pl.pallas_call(
        kern,
        out_shape=jax.ShapeDtypeStruct((B, 128), F32),
        grid=(B // TB,),
        in_specs=[
            pl.BlockSpec((TB, 96, 32), lambda i: (i, 0, 0)),
            pl.BlockSpec((384, 1024), lambda i: (0, 0)),
            pl.BlockSpec((1536, 1024), lambda i: (0, 0)),
            pl.BlockSpec((1024, 512), lambda i: (0, 0)),
            pl.BlockSpec((1024, 512), lambda i: (0, 0)),
            pl.BlockSpec((8, 512, 128), lambda i: (0, 0, 0)),
            pl.BlockSpec((128, 128), lambda i: (0, 0)),
            pl.BlockSpec((1, 1024), lambda i: (0, 0)),
            pl.BlockSpec((1, 1024), lambda i: (0, 0)),
            pl.BlockSpec((1, 128), lambda i: (0, 0)),
            pl.BlockSpec((1, 128), lambda i: (0, 0)),
        ],
        out_specs=pl.BlockSpec((TB, 128), lambda i: (i, 0)),
        scratch_shapes=[
            pltpu.VMEM((TB, 8, 5, 128), F32),
            pltpu.VMEM((TB, 5, 512), BF),
            pltpu.VMEM((TB, 5, 512), BF),
            pltpu.VMEM((TB, 4, 512), BF),
            pltpu.VMEM((TB, 4, 512), BF),
            pltpu.VMEM((TB, 8, 512), BF),
        ],
        compiler_params=pltpu.CompilerParams(
            dimension_semantics=("parallel",)),
    )(xv, wb1, wb2, s1m, s2m, w3b, w4b, b1bc, b2bc, b3p, b4p)
    return out[:, :10]


# speedup vs baseline: 1.2096x; 1.0023x over previous
import functools

import jax
import jax.numpy as jnp
from jax.experimental import pallas as pl
from jax.experimental.pallas import tpu as pltpu

BF = jnp.bfloat16
F32 = jnp.float32


def _fused_cnn_kernel(xv, wb1, wb2, s1m, s2m, w3b, w4b, b1, b2, b3, b4,
                      out_ref, sxm, sp1r0, sp1r1, sp1r2, sp1r3, sp2, *, TB):
    M4 = TB * 4

    xcv = jnp.concatenate([xv[:, 32 * c:32 * c + 32, :] for c in range(3)],
                          axis=2)
    xcv = jnp.concatenate(
        [xcv, jnp.ones((TB, 32, 1), F32), jnp.zeros((TB, 32, 31), F32)],
        axis=2)
    vt = jnp.transpose(xcv.reshape(TB, 4, 8, 128), (0, 2, 1, 3))
    sxm[:, 1:8, 0:4, :] = vt[:, 0:7, :, :]
    sxm[:, 0, 1:5, :] = vt[:, 7, 0:4, :]
    sxm[:, 0, 0, :] = jnp.zeros((TB, 128), F32)
    sxm[:, 1:8, 4, :] = jnp.zeros((TB, 7, 128), F32)

    def xp(r, q0):
        return sxm[:, r, q0:q0 + 4, :].reshape(M4, 128)

    blocks = []
    for r in range(8):
        pieces = [xp((r + d) % 8, 1 if r + d >= 8 else 0) for d in range(3)]
        blocks.append(jnp.concatenate(pieces, axis=1))
    xcat = jnp.concatenate(blocks, axis=0).astype(BF)
    a1 = jnp.dot(xcat, wb1[...], preferred_element_type=F32)
    a1 = jnp.maximum(a1, 0.0)

    hp = jnp.concatenate(
        [jnp.maximum(a1[(2 * k) * M4:(2 * k + 1) * M4],
                     a1[(2 * k + 1) * M4:(2 * k + 2) * M4]) for k in range(4)],
        axis=0)
    vr = jnp.concatenate([hp[:, 1:], hp[:, :1]], axis=1)
    m1 = jnp.maximum(hp, vr).astype(BF)
    p1 = jnp.dot(m1, s1m[...], preferred_element_type=F32).astype(BF)

    sp1r1[:, 0:4, :] = p1[0 * M4:1 * M4].reshape(TB, 4, 512)
    sp1r2[:, 0:4, :] = p1[1 * M4:2 * M4].reshape(TB, 4, 512)
    sp1r3[:, 0:4, :] = p1[2 * M4:3 * M4].reshape(TB, 4, 512)
    sp1r0[:, 1:5, :] = p1[3 * M4:4 * M4].reshape(TB, 4, 512)
    sp1r0[:, 0, :] = jnp.zeros((TB, 512), BF)
    sp1r1[:, 4, :] = jnp.zeros((TB, 512), BF)

    def sp(ref, q0):
        return ref[:, q0:q0 + 4, :].reshape(M4, 512)

    L0 = jnp.concatenate([sp(sp1r0, 0), sp(sp1r1, 0), sp(sp1r2, 0)], axis=1)
    L1 = jnp.concatenate([sp(sp1r1, 0), sp(sp1r2, 0), sp(sp1r3, 0)], axis=1)
    L2 = jnp.concatenate([sp(sp1r2, 0), sp(sp1r3, 0), sp(sp1r0, 1)], axis=1)
    L3 = jnp.concatenate([sp(sp1r3, 0), sp(sp1r0, 1), sp(sp1r1, 1)], axis=1)
    ycat = jnp.concatenate([L0, L1, L2, L3], axis=0)
    a2 = jnp.dot(ycat, wb2[...], preferred_element_type=F32)
    a2 = jnp.maximum(a2 + b2[...], 0.0)

    hp2 = jnp.concatenate(
        [jnp.maximum(a2[(2 * k) * M4:(2 * k + 1) * M4],
                     a2[(2 * k + 1) * M4:(2 * k + 2) * M4]) for k in range(2)],
        axis=0)
    vr2 = jnp.concatenate([hp2[:, 64:], hp2[:, :64]], axis=1)
    m2 = jnp.maximum(hp2, vr2).astype(BF)
    p2 = jnp.dot(m2, s2m[...], preferred_element_type=F32).astype(BF)
    sp2[:, 0:4, :] = p2[:M4].reshape(TB, 4, 512)
    sp2[:, 4:8, :] = p2[M4:].reshape(TB, 4, 512)

    h1 = b3[...].astype(F32)
    for s, h3 in enumerate((0, 2, 4, 6, 1, 3, 5, 7)):
        h1 = h1 + jnp.dot(sp2[:, s, :], w3b[h3],
                          preferred_element_type=F32)
    h1 = jnp.maximum(h1, 0.0).astype(BF)
    o = jnp.dot(h1, w4b[...], preferred_element_type=F32) + b4[...]
    out_ref[...] = o.astype(F32)


def kernel(w1p, b1p, w2p, b2p, w3p, b3p, w4p, b4p, x_nchw):
    B = x_nchw.shape[0]
    TB = 128 if x_nchw.shape[0] % 128 == 0 else 16

    w1r = w1p.reshape(3, 3, 3, 128)[:, :, :, :32]
    w2r = w2p.reshape(3, 3, 128, 128)[:, :, :32, :64]
    eyes32 = jnp.stack([jnp.eye(32, k=1 - dw, dtype=F32) for dw in range(3)])
    eyes16 = jnp.stack([jnp.eye(16, k=1 - dw, dtype=F32) for dw in range(3)])

    wb1 = jnp.einsum("hdic,dwv->hiwcv", w1r, eyes32).reshape(3, 96, 1024)
    wb1 = jnp.pad(wb1, ((0, 0), (0, 32), (0, 0)))
    b1bc = jnp.repeat(b1p.reshape(-1)[:32], 32).reshape(1, 1, 1024)
    wb1 = jnp.concatenate(
        [wb1[0:1], jnp.concatenate([wb1[1:2, :96], b1bc, wb1[1:2, 97:]],
                                   axis=1), wb1[2:3]], axis=0)
    wb1 = wb1.reshape(384, 1024).astype(BF)
    wb2 = jnp.einsum("hdic,dwv->hiwvc", w2r, eyes16).reshape(3, 512, 1024)
    wb2 = wb2.reshape(1536, 1024).astype(BF)

    ii = jax.lax.broadcasted_iota(jnp.int32, (1024, 512), 0)
    jj = jax.lax.broadcasted_iota(jnp.int32, (1024, 512), 1)
    s1m = (ii == (jj // 16) * 32 + (jj % 16) * 2).astype(BF)
    s2m = (ii == (jj // 64) * 128 + jj % 64).astype(BF)

    w3b = w3p.reshape(8, 8, 128, 128)[:, :, :64, :]
    w3b = w3b.reshape(8, 512, 128).astype(BF)
    w4b = w4p.astype(BF)

    b1bc = jnp.repeat(b1p.reshape(-1)[:32], 32).reshape(1, 1024)
    b2bc = jnp.tile(b2p.reshape(-1)[:64], 16).reshape(1, 1024)

    xv = x_nchw.reshape(B, 96, 32)

    kern = functools.partial(_fused_cnn_kernel, TB=TB)
    out = pl.pallas_call(
        kern,
        out_shape=jax.ShapeDtypeStruct((B, 128), F32),
        grid=(B // TB,),
        in_specs=[
            pl.BlockSpec((TB, 96, 32), lambda i: (i, 0, 0)),
            pl.BlockSpec((384, 1024), lambda i: (0, 0)),
            pl.BlockSpec((1536, 1024), lambda i: (0, 0)),
            pl.BlockSpec((1024, 512), lambda i: (0, 0)),
            pl.BlockSpec((1024, 512), lambda i: (0, 0)),
            pl.BlockSpec((8, 512, 128), lambda i: (0, 0, 0)),
            pl.BlockSpec((128, 128), lambda i: (0, 0)),
            pl.BlockSpec((1, 1024), lambda i: (0, 0)),
            pl.BlockSpec((1, 1024), lambda i: (0, 0)),
            pl.BlockSpec((1, 128), lambda i: (0, 0)),
            pl.BlockSpec((1, 128), lambda i: (0, 0)),
        ],
        out_specs=pl.BlockSpec((TB, 128), lambda i: (i, 0)),
        scratch_shapes=[
            pltpu.VMEM((TB, 8, 5, 128), F32),
            pltpu.VMEM((TB, 5, 512), BF),
            pltpu.VMEM((TB, 5, 512), BF),
            pltpu.VMEM((TB, 4, 512), BF),
            pltpu.VMEM((TB, 4, 512), BF),
            pltpu.VMEM((TB, 8, 512), BF),
        ],
        compiler_params=pltpu.CompilerParams(
            dimension_semantics=("parallel",)),
    )(xv, wb1, wb2, s1m, s2m, w3b, w4b, b1bc, b2bc, b3p, b4p)
    return out[:, :10]
